# baseline (speedup 1.0000x reference)
import functools

import jax
import jax.numpy as jnp
from jax import lax
from jax.experimental import pallas as pl
from jax.experimental.pallas import tpu as pltpu
from jax.experimental.pallas import tpu_sc as plsc

NCLS = 21
ALPHA = 10.0
NBOX = 4
CPB = 2
LOG2E = 1.4426950408889634
LN2 = 0.6931471805599453

BATCH = 64
NA = 8732
BW = NA * NBOX
NWORK = 32
BPW = BATCH // NWORK
L = 16


def _conf_kernel(cats_ref, gt_ref, conf_ref, n_ref, aconf_ref):
    s = pl.program_id(0)
    nb = pl.num_programs(0)
    gt = gt_ref[...]
    acc = None
    for k in range(CPB):
        c = s * CPB + k
        x = cats_ref[k]
        e2 = jax.lax.exp2(jnp.minimum(x * LOG2E, 100.0))
        sp = LN2 * jnp.log2(1.0 + e2)
        term = sp - jnp.where(gt == c, x, 0.0)
        acc = term if acc is None else acc + term

    @pl.when(s == 0)
    def _first():
        aconf_ref[...] = acc

    @pl.when(s > 0)
    def _rest():
        aconf_ref[...] += acc

    @pl.when(s == nb - 1)
    def _fin():
        maskf = (gt != NCLS - 1).astype(jnp.float32)
        conf_ref[0, 0] = jnp.sum(aconf_ref[...] * maskf)
        n_ref[0, 0] = jnp.sum(maskf)


_sc_mesh = plsc.VectorSubcoreMesh(core_axis_name="c", subcore_axis_name="s")


@functools.partial(
    pl.kernel, mesh=_sc_mesh,
    out_type=jax.ShapeDtypeStruct((NWORK, L), jnp.float32),
    scratch_types=[
        pltpu.VMEM((BW,), jnp.float32),
        pltpu.VMEM((BW,), jnp.float32),
        pltpu.VMEM((BW,), jnp.float32),
        pltpu.VMEM((NA + 16,), jnp.int32),
        pltpu.VMEM((L,), jnp.float32),
    ],
)
def _loc_sc(bbs_hbm, gtb_hbm, anc_hbm, gt_hbm, out_hbm,
            bbs_v, gtb_v, anc_v, gt_v, part_v):
    wid = lax.axis_index("s") * 2 + lax.axis_index("c")
    iota = lax.iota(jnp.int32, L)
    expand_idx = iota >> 2
    pltpu.sync_copy(anc_hbm, anc_v)
    loc = jnp.zeros((L,), jnp.float32)
    for b in range(BPW):
        bb = wid * BPW + b
        boff = pl.multiple_of(bb * BW, 8)
        pltpu.sync_copy(bbs_hbm.at[pl.ds(boff, BW)], bbs_v)
        pltpu.sync_copy(gtb_hbm.at[pl.ds(boff, BW)], gtb_v)
        r = (b % 2) * 4
        goff = pl.multiple_of(bb * NA - r, 8)
        pltpu.sync_copy(gt_hbm.at[pl.ds(goff, NA + r)],
                        gt_v.at[pl.ds(0, NA + r)])

        def body(j, acc):
            gwin = gt_v[pl.ds(j * 4 + r, L)]
            mf = jnp.where(gwin != NCLS - 1, 1.0, 0.0)
            m4 = mf.at[expand_idx].get(mode="promise_in_bounds")
            d = anc_v[pl.ds(j * L, L)] + bbs_v[pl.ds(j * L, L)] \
                - jnp.clip(gtb_v[pl.ds(j * L, L)], 0.0, 1.0)
            ad = jnp.abs(d)
            m1 = jnp.minimum(ad, 1.0)
            sl1 = 0.5 * m1 * m1 + (ad - m1)
            return acc + sl1 * m4

        loc = lax.fori_loop(0, BW // L, body, loc)
    part_v[...] = loc
    pltpu.sync_copy(part_v, out_hbm.at[wid])


def kernel(bbs_preds, cats_preds, gt_bbs, gt_cats, anchors):
    batch, n_anchors, _ = cats_preds.shape
    cats_t = jnp.transpose(cats_preds, (2, 0, 1))
    gt = gt_cats.astype(jnp.int32)

    bbs_flat = bbs_preds.reshape(-1)
    gtb_flat = gt_bbs.reshape(-1)
    anc_flat = anchors.reshape(-1)
    gt_flat = gt.reshape(-1)

    loc_parts = _loc_sc(bbs_flat, gtb_flat, anc_flat, gt_flat)

    conf, n = pl.pallas_call(
        _conf_kernel,
        grid=((NCLS - 1) // CPB,),
        in_specs=[
            pl.BlockSpec((CPB, batch, n_anchors), lambda s: (s, 0, 0)),
            pl.BlockSpec((batch, n_anchors), lambda s: (0, 0)),
        ],
        out_specs=[pl.BlockSpec((1, 1), lambda s: (0, 0),
                                memory_space=pltpu.SMEM)] * 2,
        out_shape=[jax.ShapeDtypeStruct((1, 1), jnp.float32)] * 2,
        scratch_shapes=[pltpu.VMEM((batch, n_anchors), jnp.float32)],
        compiler_params=pltpu.CompilerParams(
            dimension_semantics=("arbitrary",)),
    )(cats_t, gt)

    conf = conf[0, 0]
    n = n[0, 0]
    loc = jnp.sum(loc_parts)
    total = (conf + ALPHA * loc) / n
    return (total, loc, conf)


# speedup vs baseline: 16.5343x; 16.5343x over previous
import functools

import jax
import jax.numpy as jnp
from jax import lax
from jax.experimental import pallas as pl
from jax.experimental.pallas import tpu as pltpu
from jax.experimental.pallas import tpu_sc as plsc

NCLS = 21
ALPHA = 10.0
NBOX = 4
CPB = 2
LOG2E = 1.4426950408889634
LN2 = 0.6931471805599453

BATCH = 64
NA = 8732
L = 16
NA_SC = 8704
TAILW = NA - NA_SC
CH = 1024
SC_CHUNKS = (CH,) * 8 + (512,)


def _smooth_l1(d):
    ad = jnp.abs(d)
    m1 = jnp.minimum(ad, 1.0)
    return 0.5 * m1 * m1 + (ad - m1)


def _conf_kernel(cats_ref, gt_ref, bbs_e_ref, gtb_e_ref, anc_e_ref,
                 conf_ref, n_ref, loct_ref, aconf_ref):
    s = pl.program_id(0)
    nb = pl.num_programs(0)
    gt = gt_ref[...]
    acc = None
    for k in range(CPB):
        c = s * CPB + k
        x = cats_ref[k]
        e2 = jax.lax.exp2(jnp.minimum(x * LOG2E, 100.0))
        sp = LN2 * jnp.log2(1.0 + e2)
        term = sp - jnp.where(gt == c, x, 0.0)
        acc = term if acc is None else acc + term

    @pl.when(s == 0)
    def _first():
        aconf_ref[...] = acc
        d = anc_e_ref[...] + bbs_e_ref[...] - jnp.clip(gtb_e_ref[...], 0.0, 1.0)
        sl1 = _smooth_l1(d)[:, :, :TAILW]
        mt = (gt[:, NA - TAILW:] != NCLS - 1).astype(jnp.float32)
        loct_ref[0, 0] = jnp.sum(sl1 * mt[None])

    @pl.when(s > 0)
    def _rest():
        aconf_ref[...] += acc

    @pl.when(s == nb - 1)
    def _fin():
        maskf = (gt != NCLS - 1).astype(jnp.float32)
        conf_ref[0, 0] = jnp.sum(aconf_ref[...] * maskf)
        n_ref[0, 0] = jnp.sum(maskf)


_sc_mesh = plsc.VectorSubcoreMesh(core_axis_name="c", subcore_axis_name="s")


@functools.partial(
    pl.kernel, mesh=_sc_mesh,
    out_type=jax.ShapeDtypeStruct((32, L), jnp.float32),
    scratch_types=[
        pltpu.VMEM((8, CH), jnp.float32),
        pltpu.VMEM((8, CH), jnp.float32),
        pltpu.VMEM((8, CH), jnp.int32),
        pltpu.VMEM((NBOX, NA_SC), jnp.float32),
        pltpu.VMEM((L,), jnp.float32),
    ],
)
def _loc_sc(bbs_hbm, gtb_hbm, anc_hbm, gt_hbm, out_hbm,
            bbs_v, gtb_v, gt_v, anc_v, part_v):
    wid = lax.axis_index("s") * 2 + lax.axis_index("c")
    comp = wid >> 3
    grp = wid & 7
    pltpu.sync_copy(anc_hbm.at[:, pl.ds(0, NA_SC)], anc_v)
    loc = jnp.zeros((L,), jnp.float32)
    off = 0
    for n in SC_CHUNKS:
        pltpu.sync_copy(bbs_hbm.at[comp, pl.ds(grp * 8, 8), pl.ds(off, n)],
                        bbs_v.at[:, pl.ds(0, n)])
        pltpu.sync_copy(gtb_hbm.at[comp, pl.ds(grp * 8, 8), pl.ds(off, n)],
                        gtb_v.at[:, pl.ds(0, n)])
        pltpu.sync_copy(gt_hbm.at[pl.ds(grp * 8, 8), pl.ds(off, n)],
                        gt_v.at[:, pl.ds(0, n)])
        base = off

        def body(i, acc):
            av = anc_v[comp, pl.ds(base + i * L, L)]
            for r in range(8):
                d = av + bbs_v[r, pl.ds(i * L, L)] \
                    - jnp.clip(gtb_v[r, pl.ds(i * L, L)], 0.0, 1.0)
                sl1 = _smooth_l1(d)
                mf = jnp.where(gt_v[r, pl.ds(i * L, L)] != NCLS - 1, 1.0, 0.0)
                acc = acc + sl1 * mf
            return acc

        loc = lax.fori_loop(0, n // L, body, loc)
        off += n
    part_v[...] = loc
    pltpu.sync_copy(part_v, out_hbm.at[wid])


def kernel(bbs_preds, cats_preds, gt_bbs, gt_cats, anchors):
    batch, n_anchors, _ = cats_preds.shape
    cats_t = jnp.transpose(cats_preds, (2, 0, 1))
    gt = gt_cats.astype(jnp.int32)

    bbs_t = jnp.transpose(bbs_preds, (2, 0, 1))
    gtb_t = jnp.transpose(gt_bbs, (2, 0, 1))
    anc_t = anchors.T
    anc_3 = anc_t.reshape(NBOX, 1, n_anchors)

    conf, n, loct = pl.pallas_call(
        _conf_kernel,
        grid=((NCLS - 1) // CPB,),
        in_specs=[
            pl.BlockSpec((CPB, batch, n_anchors), lambda s: (s, 0, 0)),
            pl.BlockSpec((batch, n_anchors), lambda s: (0, 0)),
            pl.BlockSpec((NBOX, batch, 128), lambda s: (0, 0, NA_SC // 128)),
            pl.BlockSpec((NBOX, batch, 128), lambda s: (0, 0, NA_SC // 128)),
            pl.BlockSpec((NBOX, 1, 128), lambda s: (0, 0, NA_SC // 128)),
        ],
        out_specs=[pl.BlockSpec((1, 1), lambda s: (0, 0),
                                memory_space=pltpu.SMEM)] * 3,
        out_shape=[jax.ShapeDtypeStruct((1, 1), jnp.float32)] * 3,
        scratch_shapes=[pltpu.VMEM((batch, n_anchors), jnp.float32)],
        compiler_params=pltpu.CompilerParams(
            dimension_semantics=("arbitrary",)),
    )(cats_t, gt, bbs_t, gtb_t, anc_3)

    loc_parts = _loc_sc(bbs_t, gtb_t, anc_t, gt)

    conf = conf[0, 0]
    n = n[0, 0]
    loc = jnp.sum(loc_parts) + loct[0, 0]
    total = (conf + ALPHA * loc) / n
    return (total, loc, conf)


# speedup vs baseline: 18.6555x; 1.1283x over previous
import functools

import jax
import jax.numpy as jnp
from jax import lax
from jax.experimental import pallas as pl
from jax.experimental.pallas import tpu as pltpu
from jax.experimental.pallas import tpu_sc as plsc

NCLS = 21
ALPHA = 10.0
NBOX = 4
CPB = 2
LOG2E = 1.4426950408889634
LN2 = 0.6931471805599453

BATCH = 64
NA = 8732
L = 16
NA_SC = 8704
TAILW = NA - NA_SC
CH = 1024
SC_CHUNKS = (CH,) * 8 + (512,)


def _smooth_l1(d):
    ad = jnp.abs(d)
    m1 = jnp.minimum(ad, 1.0)
    return 0.5 * m1 * m1 + (ad - m1)


def _conf_kernel(cats_ref, gt_ref, bbs_e_ref, gtb_e_ref, anc_e_ref,
                 conf_ref, n_ref, loct_ref, aconf_ref):
    s = pl.program_id(0)
    nb = pl.num_programs(0)
    gt = gt_ref[...]
    acc = None
    for k in range(CPB):
        c = s * CPB + k
        x = cats_ref[k]
        e2 = jax.lax.exp2(jnp.minimum(x * LOG2E, 100.0))
        sp = LN2 * jnp.log2(1.0 + e2)
        term = sp - jnp.where(gt == c, x, 0.0)
        acc = term if acc is None else acc + term

    @pl.when(s == 0)
    def _first():
        aconf_ref[...] = acc
        d = anc_e_ref[...] + bbs_e_ref[...] - jnp.clip(gtb_e_ref[...], 0.0, 1.0)
        sl1 = _smooth_l1(d)[:, :, :TAILW]
        mt = (gt[:, NA - TAILW:] != NCLS - 1).astype(jnp.float32)
        loct_ref[0, 0] = jnp.sum(sl1 * mt[None])

    @pl.when(s > 0)
    def _rest():
        aconf_ref[...] += acc

    @pl.when(s == nb - 1)
    def _fin():
        maskf = (gt != NCLS - 1).astype(jnp.float32)
        conf_ref[0, 0] = jnp.sum(aconf_ref[...] * maskf)
        n_ref[0, 0] = jnp.sum(maskf)


_sc_mesh = plsc.VectorSubcoreMesh(core_axis_name="c", subcore_axis_name="s")


@functools.partial(
    pl.kernel, mesh=_sc_mesh,
    out_type=jax.ShapeDtypeStruct((32, L), jnp.float32),
    scratch_types=[
        pltpu.VMEM((2, 8, CH), jnp.float32),
        pltpu.VMEM((2, 8, CH), jnp.float32),
        pltpu.VMEM((2, 8, CH), jnp.int32),
        pltpu.VMEM((NBOX, NA_SC), jnp.float32),
        pltpu.VMEM((L,), jnp.float32),
        pltpu.SemaphoreType.DMA,
        pltpu.SemaphoreType.DMA,
    ],
)
def _loc_sc(bbs_hbm, gtb_hbm, anc_hbm, gt_hbm, out_hbm,
            bbs_v, gtb_v, gt_v, anc_v, part_v, sem0, sem1):
    wid = lax.axis_index("s") * 2 + lax.axis_index("c")
    comp = wid >> 3
    grp = wid & 7
    sems = (sem0, sem1)
    offs = []
    off = 0
    for n in SC_CHUNKS:
        offs.append(off)
        off += n

    def start(k):
        n = SC_CHUNKS[k]
        off = offs[k]
        buf = k % 2
        sem = sems[buf]
        cps = (
            pltpu.make_async_copy(
                bbs_hbm.at[comp, pl.ds(grp * 8, 8), pl.ds(off, n)],
                bbs_v.at[buf, :, pl.ds(0, n)], sem),
            pltpu.make_async_copy(
                gtb_hbm.at[comp, pl.ds(grp * 8, 8), pl.ds(off, n)],
                gtb_v.at[buf, :, pl.ds(0, n)], sem),
            pltpu.make_async_copy(
                gt_hbm.at[pl.ds(grp * 8, 8), pl.ds(off, n)],
                gt_v.at[buf, :, pl.ds(0, n)], sem),
        )
        for cp in cps:
            cp.start()
        return cps

    pltpu.sync_copy(anc_hbm.at[:, pl.ds(0, NA_SC)], anc_v)
    accs = [jnp.zeros((L,), jnp.float32) for _ in range(8)]
    pending = start(0)
    for k, n in enumerate(SC_CHUNKS):
        for cp in pending:
            cp.wait()
        if k + 1 < len(SC_CHUNKS):
            pending = start(k + 1)
        base = offs[k]
        buf = k % 2

        def body(i, accs):
            av = anc_v[comp, pl.ds(base + i * L, L)]
            out = []
            for r in range(8):
                d = av + bbs_v[buf, r, pl.ds(i * L, L)] \
                    - jnp.clip(gtb_v[buf, r, pl.ds(i * L, L)], 0.0, 1.0)
                sl1 = _smooth_l1(d)
                mf = jnp.where(gt_v[buf, r, pl.ds(i * L, L)] != NCLS - 1,
                               1.0, 0.0)
                out.append(accs[r] + sl1 * mf)
            return tuple(out)

        accs = lax.fori_loop(0, n // L, body, tuple(accs))
        accs = list(accs)
    loc = accs[0]
    for r in range(1, 8):
        loc = loc + accs[r]
    part_v[...] = loc
    pltpu.sync_copy(part_v, out_hbm.at[wid])


def kernel(bbs_preds, cats_preds, gt_bbs, gt_cats, anchors):
    batch, n_anchors, _ = cats_preds.shape
    cats_t = jnp.transpose(cats_preds, (2, 0, 1))
    gt = gt_cats.astype(jnp.int32)

    bbs_t = jnp.transpose(bbs_preds, (2, 0, 1))
    gtb_t = jnp.transpose(gt_bbs, (2, 0, 1))
    anc_t = anchors.T
    anc_3 = anc_t.reshape(NBOX, 1, n_anchors)

    conf, n, loct = pl.pallas_call(
        _conf_kernel,
        grid=((NCLS - 1) // CPB,),
        in_specs=[
            pl.BlockSpec((CPB, batch, n_anchors), lambda s: (s, 0, 0)),
            pl.BlockSpec((batch, n_anchors), lambda s: (0, 0)),
            pl.BlockSpec((NBOX, batch, 128), lambda s: (0, 0, NA_SC // 128)),
            pl.BlockSpec((NBOX, batch, 128), lambda s: (0, 0, NA_SC // 128)),
            pl.BlockSpec((NBOX, 1, 128), lambda s: (0, 0, NA_SC // 128)),
        ],
        out_specs=[pl.BlockSpec((1, 1), lambda s: (0, 0),
                                memory_space=pltpu.SMEM)] * 3,
        out_shape=[jax.ShapeDtypeStruct((1, 1), jnp.float32)] * 3,
        scratch_shapes=[pltpu.VMEM((batch, n_anchors), jnp.float32)],
        compiler_params=pltpu.CompilerParams(
            dimension_semantics=("arbitrary",)),
    )(cats_t, gt, bbs_t, gtb_t, anc_3)

    loc_parts = _loc_sc(bbs_t, gtb_t, anc_t, gt)

    conf = conf[0, 0]
    n = n[0, 0]
    loc = jnp.sum(loc_parts) + loct[0, 0]
    total = (conf + ALPHA * loc) / n
    return (total, loc, conf)


# speedup vs baseline: 23.7533x; 1.2733x over previous
import jax
import jax.numpy as jnp
from jax.experimental import pallas as pl
from jax.experimental.pallas import tpu as pltpu

NCLS = 21
ALPHA = 10.0
NBOX = 4
CPB = 4
LOG2E = 1.4426950408889634
LN2 = 0.6931471805599453


def _loss_kernel(cats_ref, bbs_ref, gtb_ref, anc_ref, gt_ref,
                 conf_ref, loc_ref, n_ref, total_ref, aconf_ref, abox_ref):
    s = pl.program_id(0)
    nb = pl.num_programs(0)

    @pl.when(s == 0)
    def _box():
        acc = None
        for c in range(NBOX):
            d = anc_ref[c] + bbs_ref[c] - jnp.clip(gtb_ref[c], 0.0, 1.0)
            ad = jnp.abs(d)
            sl1 = jnp.where(ad < 1.0, 0.5 * d * d, ad - 0.5)
            acc = sl1 if acc is None else acc + sl1
        abox_ref[...] = acc

    @pl.when(s > 0)
    def _cls():
        gt = gt_ref[...]
        acc = None
        for k in range(CPB):
            c = (s - 1) * CPB + k
            x = cats_ref[k]
            e2 = jax.lax.exp2(jnp.minimum(x * LOG2E, 100.0))
            sp = LN2 * jnp.log2(1.0 + e2)
            term = sp - jnp.where(gt == c, x, 0.0)
            acc = term if acc is None else acc + term

        @pl.when(s == 1)
        def _first():
            aconf_ref[...] = acc

        @pl.when(s > 1)
        def _rest():
            aconf_ref[...] += acc

    @pl.when(s == nb - 1)
    def _fin():
        maskf = (gt_ref[...] != NCLS - 1).astype(jnp.float32)
        conf = jnp.sum(aconf_ref[...] * maskf)
        loc = jnp.sum(abox_ref[...] * maskf)
        n = jnp.sum(maskf)
        conf_ref[0, 0] = conf
        loc_ref[0, 0] = loc
        n_ref[0, 0] = n
        total_ref[0, 0] = (conf + ALPHA * loc) / n


def kernel(bbs_preds, cats_preds, gt_bbs, gt_cats, anchors):
    batch, n_anchors, _ = cats_preds.shape
    cats_t = jnp.transpose(cats_preds, (2, 0, 1))
    bbs_t = jnp.transpose(bbs_preds, (2, 0, 1))
    gtb_t = jnp.transpose(gt_bbs, (2, 0, 1))
    anc_t = anchors.T.reshape(NBOX, 1, n_anchors)
    gt = gt_cats.astype(jnp.int32)

    grid = (1 + (NCLS - 1) // CPB,)
    conf, loc, n, total = pl.pallas_call(
        _loss_kernel,
        grid=grid,
        in_specs=[
            pl.BlockSpec((CPB, batch, n_anchors),
                         lambda s: (jnp.maximum(s - 1, 0), 0, 0)),
            pl.BlockSpec((NBOX, batch, n_anchors), lambda s: (0, 0, 0)),
            pl.BlockSpec((NBOX, batch, n_anchors), lambda s: (0, 0, 0)),
            pl.BlockSpec((NBOX, 1, n_anchors), lambda s: (0, 0, 0)),
            pl.BlockSpec((batch, n_anchors), lambda s: (0, 0)),
        ],
        out_specs=[pl.BlockSpec((1, 1), lambda s: (0, 0),
                                memory_space=pltpu.SMEM)] * 4,
        out_shape=[jax.ShapeDtypeStruct((1, 1), jnp.float32)] * 4,
        scratch_shapes=[pltpu.VMEM((batch, n_anchors), jnp.float32),
                        pltpu.VMEM((batch, n_anchors), jnp.float32)],
        compiler_params=pltpu.CompilerParams(
            dimension_semantics=("arbitrary",)),
    )(cats_t, bbs_t, gtb_t, anc_t, gt)
    return (total[0, 0], loc[0, 0], conf[0, 0])


# speedup vs baseline: 24.0866x; 1.0140x over previous
import jax
import jax.numpy as jnp
from jax.experimental import pallas as pl
from jax.experimental.pallas import tpu as pltpu

NCLS = 21
ALPHA = 10.0
NBOX = 4
CPB = 2
LOG2E = 1.4426950408889634
LN2 = 0.6931471805599453


def _loss_kernel(cats_ref, bbs_ref, gtb_ref, anc_ref, gt_ref,
                 conf_ref, loc_ref, n_ref, total_ref, aconf_ref, abox_ref):
    s = pl.program_id(0)
    nb = pl.num_programs(0)

    @pl.when(s == 0)
    def _box():
        acc = None
        for c in range(NBOX):
            d = anc_ref[c] + bbs_ref[c] - jnp.clip(gtb_ref[c], 0.0, 1.0)
            ad = jnp.abs(d)
            sl1 = jnp.where(ad < 1.0, 0.5 * d * d, ad - 0.5)
            acc = sl1 if acc is None else acc + sl1
        abox_ref[...] = acc

    @pl.when(s > 0)
    def _cls():
        gt = gt_ref[...]
        acc = None
        for k in range(CPB):
            c = (s - 1) * CPB + k
            x = cats_ref[k]
            e2 = jax.lax.exp2(jnp.minimum(x * LOG2E, 100.0))
            sp = LN2 * jnp.log2(1.0 + e2)
            term = sp - jnp.where(gt == c, x, 0.0)
            acc = term if acc is None else acc + term

        @pl.when(s == 1)
        def _first():
            aconf_ref[...] = acc

        @pl.when(s > 1)
        def _rest():
            aconf_ref[...] += acc

    @pl.when(s == nb - 1)
    def _fin():
        maskf = (gt_ref[...] != NCLS - 1).astype(jnp.float32)
        conf = jnp.sum(aconf_ref[...] * maskf)
        loc = jnp.sum(abox_ref[...] * maskf)
        n = jnp.sum(maskf)
        conf_ref[0, 0] = conf
        loc_ref[0, 0] = loc
        n_ref[0, 0] = n
        total_ref[0, 0] = (conf + ALPHA * loc) / n


def kernel(bbs_preds, cats_preds, gt_bbs, gt_cats, anchors):
    batch, n_anchors, _ = cats_preds.shape
    cats_t = jnp.transpose(cats_preds, (2, 0, 1))
    bbs_t = jnp.transpose(bbs_preds, (2, 0, 1))
    gtb_t = jnp.transpose(gt_bbs, (2, 0, 1))
    anc_t = anchors.T.reshape(NBOX, 1, n_anchors)
    gt = gt_cats.astype(jnp.int32)

    grid = (1 + (NCLS - 1) // CPB,)
    conf, loc, n, total = pl.pallas_call(
        _loss_kernel,
        grid=grid,
        in_specs=[
            pl.BlockSpec((CPB, batch, n_anchors),
                         lambda s: (jnp.maximum(s - 1, 0), 0, 0)),
            pl.BlockSpec((NBOX, batch, n_anchors), lambda s: (0, 0, 0)),
            pl.BlockSpec((NBOX, batch, n_anchors), lambda s: (0, 0, 0)),
            pl.BlockSpec((NBOX, 1, n_anchors), lambda s: (0, 0, 0)),
            pl.BlockSpec((batch, n_anchors), lambda s: (0, 0)),
        ],
        out_specs=[pl.BlockSpec((1, 1), lambda s: (0, 0),
                                memory_space=pltpu.SMEM)] * 4,
        out_shape=[jax.ShapeDtypeStruct((1, 1), jnp.float32)] * 4,
        scratch_shapes=[pltpu.VMEM((batch, n_anchors), jnp.float32),
                        pltpu.VMEM((batch, n_anchors), jnp.float32)],
        compiler_params=pltpu.CompilerParams(
            dimension_semantics=("arbitrary",)),
    )(cats_t, bbs_t, gtb_t, anc_t, gt)
    return (total[0, 0], loc[0, 0], conf[0, 0])
